# baseline (speedup 1.0000x reference)
import functools

import jax
import jax.numpy as jnp
from jax import lax
from jax.experimental import pallas as pl
from jax.experimental.pallas import tpu as pltpu
from jax.experimental.pallas import tpu_sc as plsc

N_NODES = 10000
N_PAD = 10240
N_EDGES = 320000
D = 128
NC = 2
NS = 16
CHUNK = 64
GROUP = 16
TOTAL_GROUPS = 320
E_PAD = TOTAL_GROUPS * GROUP * CHUNK
N0_GROUPS = 6
N1_GROUPS = 20 - N0_GROUPS
ROWS_PER_TILE = N_PAD // NS


def _sc_accumulate(node_emb, rel_emb, heads3d, tails3d):
  mesh = plsc.VectorSubcoreMesh(core_axis_name="c", subcore_axis_name="s")

  @functools.partial(
      pl.kernel,
      out_type=(
          jax.ShapeDtypeStruct((NC * N_PAD, D), jnp.float32),
          jax.ShapeDtypeStruct((NC * N_PAD, D), jnp.float32),
      ),
      mesh=mesh,
      scratch_types=[
          pltpu.VMEM_SHARED((N_PAD, D), jnp.float32),
          pltpu.VMEM((GROUP, CHUNK), jnp.int32),
          pltpu.VMEM((GROUP, CHUNK), jnp.int32),
          pltpu.VMEM((CHUNK, D), jnp.float32),
          pltpu.VMEM((CHUNK, D), jnp.float32),
          pltpu.VMEM((CHUNK, D), jnp.float32),
          pltpu.VMEM((CHUNK, D), jnp.float32),
      ] + [pltpu.SemaphoreType.DMA] * 8,
  )
  def k(node_hbm, rel_hbm, heads_hbm, tails_hbm, psums_hbm, pcounts_hbm,
        acc_sh, heads_i, tails_i, gbuf0, gbuf1, rbuf0, rbuf1,
        sg0, sg1, sr0, sr1, ssg0, ssg1, ssr0, ssr1):
    c = lax.axis_index("c")
    s = lax.axis_index("s")
    zeros16 = jnp.zeros((16,), jnp.float32)
    ones16 = jnp.ones((16,), jnp.float32)
    gb = (gbuf0, gbuf1)
    rb = (rbuf0, rbuf1)
    sem_g = (sg0, sg1)
    sem_r = (sr0, sr1)
    sem_sg = (ssg0, ssg1)
    sem_sr = (ssr0, ssr1)

    g_begin = jnp.where(c == 0, s * N0_GROUPS,
                        NS * N0_GROUPS + s * N1_GROUPS)
    n_groups = jnp.where(c == 0, N0_GROUPS, N1_GROUPS)

    def fill(buf, val16):
      def body(i, _):
        for cc in range(D // 16):
          buf[i, pl.ds(cc * 16, 16)] = val16
        return 0
      lax.fori_loop(0, CHUNK, body, 0)

    fill(gbuf0, zeros16)

    def zero_own_slice():
      for kk in range(ROWS_PER_TILE // CHUNK):
        base = s * ROWS_PER_TILE + kk * CHUNK
        pltpu.sync_copy(gbuf0, acc_sh.at[pl.ds(base, CHUNK)])

    def write_own_slice(out_hbm, bounce):
      for kk in range(ROWS_PER_TILE // CHUNK):
        base = s * ROWS_PER_TILE + kk * CHUNK
        pltpu.sync_copy(acc_sh.at[pl.ds(base, CHUNK)], bounce)
        pltpu.sync_copy(bounce, out_hbm.at[pl.ds(c * N_PAD + base, CHUNK)])

    zero_own_slice()
    plsc.subcore_barrier()

    def rel_slice(gg, j):
      ebase = (gg * GROUP + j) * CHUNK
      return rel_hbm.at[pl.ds(jnp.minimum(ebase, N_EDGES - CHUNK), CHUNK)]

    def group_a(g, _):
      gg = g_begin + g
      pltpu.sync_copy(heads_hbm.at[gg], heads_i)
      pltpu.sync_copy(tails_hbm.at[gg], tails_i)
      g_d = [None, None]
      r_d = [None, None]
      sg_d = [None, None]
      sr_d = [None, None]
      g_d[0] = pltpu.async_copy(node_hbm.at[heads_i.at[0]], gb[0], sem_g[0])
      r_d[0] = pltpu.async_copy(rel_slice(gg, 0), rb[0], sem_r[0])
      for j in range(GROUP):
        p = j % 2
        q = 1 - p
        trow = tails_i.at[j]
        g_d[p].wait()
        sg_d[p] = pltpu.async_copy(gb[p], acc_sh.at[trow], sem_sg[p],
                                   add=True)
        r_d[p].wait()
        sr_d[p] = pltpu.async_copy(rb[p], acc_sh.at[trow], sem_sr[p],
                                   add=True)
        if j + 1 < GROUP:
          if sg_d[q] is not None:
            sg_d[q].wait()
          g_d[q] = pltpu.async_copy(node_hbm.at[heads_i.at[j + 1]], gb[q],
                                    sem_g[q])
          if sr_d[q] is not None:
            sr_d[q].wait()
          r_d[q] = pltpu.async_copy(rel_slice(gg, j + 1), rb[q], sem_r[q])
      sg_d[0].wait()
      sg_d[1].wait()
      sr_d[0].wait()
      sr_d[1].wait()
      return 0

    lax.fori_loop(0, n_groups, group_a, 0)

    plsc.subcore_barrier()
    write_own_slice(psums_hbm, rbuf0)
    fill(gbuf0, zeros16)
    zero_own_slice()
    plsc.subcore_barrier()

    fill(rbuf0, ones16)

    def group_b(g, _):
      gg = g_begin + g
      pltpu.sync_copy(tails_hbm.at[gg], tails_i)
      descs = []
      for j in range(GROUP):
        descs.append(pltpu.async_copy(rbuf0, acc_sh.at[tails_i.at[j]],
                                      ssr0, add=True))
        if len(descs) >= 4:
          descs.pop(0).wait()
      for d in descs:
        d.wait()
      return 0

    lax.fori_loop(0, n_groups, group_b, 0)

    plsc.subcore_barrier()
    write_own_slice(pcounts_hbm, gbuf0)

  return k(node_emb, rel_emb, heads3d, tails3d)


_TC_ROWS = 2000


def _tc_combine_body(ps_ref, pc_ref, ne_ref, m_ref, out_ref):
  ps = ps_ref[...]
  pc = pc_ref[...]
  sums = ps[0] + ps[1]
  cnt = pc[0, :, 0:1] + pc[1, :, 0:1]
  mean = sums / cnt
  out_ref[...] = jnp.where(m_ref[...] > 0, mean, ne_ref[...])


def _tc_combine(psums, pcounts, node_emb, mask):
  grid = (N_NODES // _TC_ROWS,)
  return pl.pallas_call(
      _tc_combine_body,
      grid=grid,
      in_specs=[
          pl.BlockSpec((NC, _TC_ROWS, D), lambda i: (0, i, 0)),
          pl.BlockSpec((NC, _TC_ROWS, D), lambda i: (0, i, 0)),
          pl.BlockSpec((_TC_ROWS, D), lambda i: (i, 0)),
          pl.BlockSpec((_TC_ROWS, 1), lambda i: (i, 0)),
      ],
      out_specs=pl.BlockSpec((_TC_ROWS, D), lambda i: (i, 0)),
      out_shape=jax.ShapeDtypeStruct((N_NODES, D), jnp.float32),
  )(psums, pcounts, node_emb, mask)


@jax.jit
def kernel(node_embeddings, relation_embeddings, edge_index, node_is_cvt):
  hpad = jnp.zeros((E_PAD - N_EDGES,), jnp.int32)
  tpad = jnp.full((E_PAD - N_EDGES,), N_NODES, jnp.int32)
  heads3d = jnp.concatenate([edge_index[0], hpad]).reshape(
      TOTAL_GROUPS, GROUP, CHUNK)
  tails3d = jnp.concatenate([edge_index[1], tpad]).reshape(
      TOTAL_GROUPS, GROUP, CHUNK)
  psums, pcounts = _sc_accumulate(node_embeddings, relation_embeddings,
                                  heads3d, tails3d)
  mask = node_is_cvt.astype(jnp.float32).reshape(N_NODES, 1)
  return _tc_combine(psums.reshape(NC, N_PAD, D),
                     pcounts.reshape(NC, N_PAD, D),
                     node_embeddings, mask)


# speedup vs baseline: 1.3475x; 1.3475x over previous
import functools

import jax
import jax.numpy as jnp
from jax import lax
from jax.experimental import pallas as pl
from jax.experimental.pallas import tpu as pltpu
from jax.experimental.pallas import tpu_sc as plsc

N_NODES = 10000
N_PAD = 10240
N_EDGES = 320000
D = 128
NC = 2
NS = 16
CHUNK = 64
GROUP = 16
TOTAL_GROUPS = 320
E_PAD = TOTAL_GROUPS * GROUP * CHUNK
N0_GROUPS = 14
N1_GROUPS = 20 - N0_GROUPS
ROWS_PER_TILE = N_PAD // NS


def _sc_accumulate(node_emb, rel_emb, heads3d, tails3d):
  mesh = plsc.VectorSubcoreMesh(core_axis_name="c", subcore_axis_name="s")

  @functools.partial(
      pl.kernel,
      out_type=(
          jax.ShapeDtypeStruct((NC * N_PAD, D), jnp.float32),
          jax.ShapeDtypeStruct((NC * N_PAD, D), jnp.float32),
      ),
      mesh=mesh,
      scratch_types=[
          pltpu.VMEM_SHARED((N_PAD, D), jnp.float32),
          pltpu.VMEM((GROUP, CHUNK), jnp.int32),
          pltpu.VMEM((GROUP, CHUNK), jnp.int32),
          pltpu.VMEM((CHUNK, D), jnp.float32),
          pltpu.VMEM((CHUNK, D), jnp.float32),
          pltpu.VMEM((CHUNK, D), jnp.float32),
          pltpu.VMEM((CHUNK, D), jnp.float32),
      ] + [pltpu.SemaphoreType.DMA] * 8,
  )
  def k(node_hbm, rel_hbm, heads_hbm, tails_hbm, psums_hbm, pcounts_hbm,
        acc_sh, heads_i, tails_i, gbuf0, gbuf1, rbuf0, rbuf1,
        sg0, sg1, sr0, sr1, ssg0, ssg1, ssr0, ssr1):
    c = lax.axis_index("c")
    s = lax.axis_index("s")
    zeros16 = jnp.zeros((16,), jnp.float32)
    ones16 = jnp.ones((16,), jnp.float32)
    gb = (gbuf0, gbuf1)
    rb = (rbuf0, rbuf1)
    sem_g = (sg0, sg1)
    sem_r = (sr0, sr1)
    sem_sg = (ssg0, ssg1)
    sem_sr = (ssr0, ssr1)

    g_begin = jnp.where(c == 0, s * N0_GROUPS,
                        NS * N0_GROUPS + s * N1_GROUPS)
    n_groups = jnp.where(c == 0, N0_GROUPS, N1_GROUPS)

    def fill(buf, val16):
      def body(i, _):
        for cc in range(D // 16):
          buf[i, pl.ds(cc * 16, 16)] = val16
        return 0
      lax.fori_loop(0, CHUNK, body, 0)

    fill(gbuf0, zeros16)

    def zero_own_slice():
      for kk in range(ROWS_PER_TILE // CHUNK):
        base = s * ROWS_PER_TILE + kk * CHUNK
        pltpu.sync_copy(gbuf0, acc_sh.at[pl.ds(base, CHUNK)])

    def write_own_slice(out_hbm, bounce):
      for kk in range(ROWS_PER_TILE // CHUNK):
        base = s * ROWS_PER_TILE + kk * CHUNK
        pltpu.sync_copy(acc_sh.at[pl.ds(base, CHUNK)], bounce)
        pltpu.sync_copy(bounce, out_hbm.at[pl.ds(c * N_PAD + base, CHUNK)])

    zero_own_slice()
    plsc.subcore_barrier()

    def rel_slice(gg, j):
      ebase = (gg * GROUP + j) * CHUNK
      return rel_hbm.at[pl.ds(jnp.minimum(ebase, N_EDGES - CHUNK), CHUNK)]

    def group_a(g, _):
      gg = g_begin + g
      pltpu.sync_copy(heads_hbm.at[gg], heads_i)
      pltpu.sync_copy(tails_hbm.at[gg], tails_i)
      g_d = [None, None]
      r_d = [None, None]
      sg_d = [None, None]
      sr_d = [None, None]
      g_d[0] = pltpu.async_copy(node_hbm.at[heads_i.at[0]], gb[0], sem_g[0])
      r_d[0] = pltpu.async_copy(rel_slice(gg, 0), rb[0], sem_r[0])
      for j in range(GROUP):
        p = j % 2
        q = 1 - p
        trow = tails_i.at[j]
        g_d[p].wait()
        sg_d[p] = pltpu.async_copy(gb[p], acc_sh.at[trow], sem_sg[p],
                                   add=True)
        r_d[p].wait()
        sr_d[p] = pltpu.async_copy(rb[p], acc_sh.at[trow], sem_sr[p],
                                   add=True)
        if j + 1 < GROUP:
          if sg_d[q] is not None:
            sg_d[q].wait()
          g_d[q] = pltpu.async_copy(node_hbm.at[heads_i.at[j + 1]], gb[q],
                                    sem_g[q])
          if sr_d[q] is not None:
            sr_d[q].wait()
          r_d[q] = pltpu.async_copy(rel_slice(gg, j + 1), rb[q], sem_r[q])
      sg_d[0].wait()
      sg_d[1].wait()
      sr_d[0].wait()
      sr_d[1].wait()
      return 0

    lax.fori_loop(0, n_groups, group_a, 0)

    plsc.subcore_barrier()
    write_own_slice(psums_hbm, rbuf0)
    fill(gbuf0, zeros16)
    zero_own_slice()
    plsc.subcore_barrier()

    fill(rbuf0, ones16)

    def group_b(g, _):
      gg = g_begin + g
      pltpu.sync_copy(tails_hbm.at[gg], tails_i)
      descs = []
      for j in range(GROUP):
        descs.append(pltpu.async_copy(rbuf0, acc_sh.at[tails_i.at[j]],
                                      ssr0, add=True))
        if len(descs) >= 4:
          descs.pop(0).wait()
      for d in descs:
        d.wait()
      return 0

    lax.fori_loop(0, n_groups, group_b, 0)

    plsc.subcore_barrier()
    write_own_slice(pcounts_hbm, gbuf0)

  return k(node_emb, rel_emb, heads3d, tails3d)


_TC_ROWS = 2000


def _tc_combine_body(ps_ref, pc_ref, ne_ref, m_ref, out_ref):
  ps = ps_ref[...]
  pc = pc_ref[...]
  sums = ps[0] + ps[1]
  cnt = pc[0, :, 0:1] + pc[1, :, 0:1]
  mean = sums / cnt
  out_ref[...] = jnp.where(m_ref[...] > 0, mean, ne_ref[...])


def _tc_combine(psums, pcounts, node_emb, mask):
  grid = (N_NODES // _TC_ROWS,)
  return pl.pallas_call(
      _tc_combine_body,
      grid=grid,
      in_specs=[
          pl.BlockSpec((NC, _TC_ROWS, D), lambda i: (0, i, 0)),
          pl.BlockSpec((NC, _TC_ROWS, D), lambda i: (0, i, 0)),
          pl.BlockSpec((_TC_ROWS, D), lambda i: (i, 0)),
          pl.BlockSpec((_TC_ROWS, 1), lambda i: (i, 0)),
      ],
      out_specs=pl.BlockSpec((_TC_ROWS, D), lambda i: (i, 0)),
      out_shape=jax.ShapeDtypeStruct((N_NODES, D), jnp.float32),
  )(psums, pcounts, node_emb, mask)


@jax.jit
def kernel(node_embeddings, relation_embeddings, edge_index, node_is_cvt):
  hpad = jnp.zeros((E_PAD - N_EDGES,), jnp.int32)
  tpad = jnp.full((E_PAD - N_EDGES,), N_NODES, jnp.int32)
  heads3d = jnp.concatenate([edge_index[0], hpad]).reshape(
      TOTAL_GROUPS, GROUP, CHUNK)
  tails3d = jnp.concatenate([edge_index[1], tpad]).reshape(
      TOTAL_GROUPS, GROUP, CHUNK)
  psums, pcounts = _sc_accumulate(node_embeddings, relation_embeddings,
                                  heads3d, tails3d)
  mask = node_is_cvt.astype(jnp.float32).reshape(N_NODES, 1)
  return _tc_combine(psums.reshape(NC, N_PAD, D),
                     pcounts.reshape(NC, N_PAD, D),
                     node_embeddings, mask)
